# baseline (speedup 1.0000x reference)
import functools

import jax
import jax.numpy as jnp
from jax import lax
from jax.experimental import pallas as pl
from jax.experimental.pallas import tpu as pltpu
from jax.experimental.pallas import tpu_sc as plsc

H, W, C = 720, 1280, 512
NV = 2562
HW = H * W
KCH = 3
NW = 32
PPT = HW // NW
VPT = PPT // 16
TBL = 4096
VPW = TBL // NW

_MESH = plsc.VectorSubcoreMesh(core_axis_name="c", subcore_axis_name="s")
_PARAMS = pltpu.CompilerParams(needs_layout_passes=False)
_PARAMS_B = pltpu.CompilerParams(needs_layout_passes=False,
                                 use_tc_tiling_on_sc=True)


@functools.partial(
    pl.kernel,
    mesh=_MESH,
    compiler_params=_PARAMS,
    out_type=jax.ShapeDtypeStruct((NW * TBL,), jnp.int32),
    scratch_types=[
        pltpu.VMEM((PPT * KCH,), jnp.int32),
        pltpu.VMEM((TBL,), jnp.int32),
        pltpu.VMEM((16,), jnp.int32),
    ],
)
def _winner_keys(ids_hbm, tbl_hbm, chunk, tbl, tmp):
    w = lax.axis_index("s") * 2 + lax.axis_index("c")
    pltpu.sync_copy(ids_hbm.at[pl.ds(w * PPT * KCH, PPT * KCH)], chunk)

    lane = lax.iota(jnp.int32, 16)
    shift_up = jnp.minimum(lane + 1, 15)
    is_top = lane == 15

    def init(i, carry):
        tbl[pl.ds(i * 16, 16)] = jnp.full((16,), -1, jnp.int32)
        return carry

    lax.fori_loop(0, TBL // 16, init, 0)

    pix_base = w * PPT
    for k in range(KCH):
        def body(j, carry, k=k):
            ids = plsc.load_gather(chunk, [j * 48 + lane * 3 + k])
            comp = ids * 16 + lane
            scomp, slane = plsc.sort_key_val(comp, lane)
            sid = lax.shift_right_logical(scomp, 4)
            tmp[...] = sid
            nxt = plsc.load_gather(tmp, [shift_up])
            last_of_group = (sid != nxt) | is_top
            key = (k * HW + pix_base + j * 16) + slane
            plsc.store_scatter(tbl, [sid], key, mask=last_of_group)
            return carry

        lax.fori_loop(0, VPT, body, 0)

    pltpu.sync_copy(tbl, tbl_hbm.at[pl.ds(w * TBL, TBL)])


@functools.partial(
    pl.kernel,
    mesh=_MESH,
    compiler_params=_PARAMS_B,
    out_type=jax.ShapeDtypeStruct((TBL, C), jnp.float32),
    scratch_types=[
        pltpu.VMEM((NW, VPW), jnp.int32),
        pltpu.VMEM((VPW,), jnp.int32),
        pltpu.VMEM((VPW,), jnp.int32),
        pltpu.VMEM((VPW, C), jnp.float32),
        pltpu.SemaphoreType.DMA,
    ],
)
def _gather_rows(tbl_hbm, img_hbm, out_hbm, tb, win, idxv, rows, sem):
    w = lax.axis_index("s") * 2 + lax.axis_index("c")
    vbase = w * VPW
    copies = [
        pltpu.async_copy(tbl_hbm.at[pl.ds(t * TBL + vbase, VPW)], tb.at[t], sem)
        for t in range(NW)
    ]
    for cp in copies:
        cp.wait()

    lane = lax.iota(jnp.int32, 16)
    miss_lanes = jnp.zeros((16,), jnp.int32)
    for v in range(VPW // 16):
        m = tb[0, pl.ds(v * 16, 16)]
        for t in range(1, NW):
            m = jnp.maximum(m, tb[t, pl.ds(v * 16, 16)])
        win[pl.ds(v * 16, 16)] = m
        hit = m >= 0
        vid = vbase + v * 16 + lane
        idxv[pl.ds(v * 16, 16)] = jnp.where(hit, lax.rem(m, HW), vid)
        real_miss = jnp.logical_and(jnp.logical_not(hit), vid < NV)
        miss_lanes = miss_lanes + jnp.where(real_miss, 1, 0)

    pltpu.async_copy(img_hbm.at[idxv], rows, sem).wait()

    n_miss = jnp.max(miss_lanes)

    @pl.when(n_miss > 0)
    def _patch_misses():
        def fix(r, carry):
            rs = jnp.full((16,), 0, jnp.int32) + r
            wk = plsc.load_gather(win, [rs])
            is_miss = wk < 0
            for cb in range(C // 16):
                ci = lane + cb * 16
                seg = plsc.load_gather(rows, [rs, ci])
                plsc.store_scatter(rows, [rs, ci],
                                   jnp.where(is_miss, 0.5, seg))
            return carry

        lax.fori_loop(0, VPW, fix, 0)

    pltpu.sync_copy(rows, out_hbm.at[pl.ds(vbase, VPW)])


def kernel(vert_ids, image_array):
    ids_flat = vert_ids.reshape(HW * KCH)
    img = image_array.reshape(HW, C)
    tbls = _winner_keys(ids_flat)
    padded = _gather_rows(tbls, img)
    return padded[:NV][None]


# speedup vs baseline: 15.3580x; 15.3580x over previous
import functools

import jax
import jax.numpy as jnp
from jax import lax
from jax.experimental import pallas as pl
from jax.experimental.pallas import tpu as pltpu
from jax.experimental.pallas import tpu_sc as plsc

H, W, C = 720, 1280, 512
NV = 2562
HW = H * W
KCH = 3
NW = 32
PPT = HW // NW
VPT = PPT // 16
TBL = 4096
VPW = TBL // NW

_MESH = plsc.VectorSubcoreMesh(core_axis_name="c", subcore_axis_name="s")
_PARAMS = pltpu.CompilerParams(needs_layout_passes=False)
_PARAMS_B = pltpu.CompilerParams(needs_layout_passes=False,
                                 use_tc_tiling_on_sc=True)


@functools.partial(
    pl.kernel,
    mesh=_MESH,
    compiler_params=_PARAMS,
    out_type=jax.ShapeDtypeStruct((NW * TBL,), jnp.int32),
    scratch_types=[
        pltpu.VMEM((PPT,), jnp.int32),
        pltpu.VMEM((TBL,), jnp.int32),
        pltpu.VMEM((16,), jnp.int32),
    ],
)
def _winner_keys(ids_hbm, tbl_hbm, chunk, tbl, tmp):
    w = lax.axis_index("s") * 2 + lax.axis_index("c")

    lane = lax.iota(jnp.int32, 16)
    shift_up = jnp.minimum(lane + 1, 15)
    is_top = lane == 15

    def init(i, carry):
        tbl[pl.ds(i * 16, 16)] = jnp.full((16,), -1, jnp.int32)
        return carry

    lax.fori_loop(0, TBL // 16, init, 0)

    pix_base = w * PPT
    for k in range(KCH):
        pltpu.sync_copy(ids_hbm.at[pl.ds(k * HW + pix_base, PPT)], chunk)

        def body(j, carry, k=k):
            ids = chunk[pl.ds(j * 16, 16)]
            comp = ids * 16 + lane
            scomp, slane = plsc.sort_key_val(comp, lane)
            sid = lax.shift_right_logical(scomp, 4)
            tmp[...] = sid
            nxt = plsc.load_gather(tmp, [shift_up])
            last_of_group = (sid != nxt) | is_top
            key = (k * HW + pix_base + j * 16) + slane
            plsc.store_scatter(tbl, [sid], key, mask=last_of_group)
            return carry

        lax.fori_loop(0, VPT, body, 0)

    pltpu.sync_copy(tbl, tbl_hbm.at[pl.ds(w * TBL, TBL)])


@functools.partial(
    pl.kernel,
    mesh=_MESH,
    compiler_params=_PARAMS_B,
    out_type=jax.ShapeDtypeStruct((TBL, C), jnp.float32),
    scratch_types=[
        pltpu.VMEM((NW, VPW), jnp.int32),
        pltpu.VMEM((VPW,), jnp.int32),
        pltpu.VMEM((VPW,), jnp.int32),
        pltpu.VMEM((VPW, C), jnp.float32),
        pltpu.SemaphoreType.DMA,
    ],
)
def _gather_rows(tbl_hbm, img_hbm, out_hbm, tb, win, idxv, rows, sem):
    w = lax.axis_index("s") * 2 + lax.axis_index("c")
    vbase = w * VPW
    copies = [
        pltpu.async_copy(tbl_hbm.at[pl.ds(t * TBL + vbase, VPW)], tb.at[t], sem)
        for t in range(NW)
    ]
    for cp in copies:
        cp.wait()

    lane = lax.iota(jnp.int32, 16)
    miss_lanes = jnp.zeros((16,), jnp.int32)
    for v in range(VPW // 16):
        m = tb[0, pl.ds(v * 16, 16)]
        for t in range(1, NW):
            m = jnp.maximum(m, tb[t, pl.ds(v * 16, 16)])
        win[pl.ds(v * 16, 16)] = m
        hit = m >= 0
        vid = vbase + v * 16 + lane
        idxv[pl.ds(v * 16, 16)] = jnp.where(hit, lax.rem(m, HW), vid)
        real_miss = jnp.logical_and(jnp.logical_not(hit), vid < NV)
        miss_lanes = miss_lanes + jnp.where(real_miss, 1, 0)

    pltpu.async_copy(img_hbm.at[idxv], rows, sem).wait()

    n_miss = jnp.max(miss_lanes)

    @pl.when(n_miss > 0)
    def _patch_misses():
        def fix(r, carry):
            rs = jnp.full((16,), 0, jnp.int32) + r
            wk = plsc.load_gather(win, [rs])
            is_miss = wk < 0
            for cb in range(C // 16):
                ci = lane + cb * 16
                seg = plsc.load_gather(rows, [rs, ci])
                plsc.store_scatter(rows, [rs, ci],
                                   jnp.where(is_miss, 0.5, seg))
            return carry

        lax.fori_loop(0, VPW, fix, 0)

    pltpu.sync_copy(rows, out_hbm.at[pl.ds(vbase, VPW)])


def kernel(vert_ids, image_array):
    ids_flat = jnp.transpose(vert_ids, (2, 0, 1)).reshape(KCH * HW)
    img = image_array.reshape(HW, C)
    tbls = _winner_keys(ids_flat)
    padded = _gather_rows(tbls, img)
    return padded[:NV][None]


# speedup vs baseline: 17.3834x; 1.1319x over previous
import functools

import jax
import jax.numpy as jnp
from jax import lax
from jax.experimental import pallas as pl
from jax.experimental.pallas import tpu as pltpu
from jax.experimental.pallas import tpu_sc as plsc

H, W, C = 720, 1280, 512
NV = 2562
HW = H * W
KCH = 3
NW = 32
PPT = HW // NW
VPT = PPT // 16
TBL = 4096
VPW = TBL // NW
UNROLL = 4

_MESH = plsc.VectorSubcoreMesh(core_axis_name="c", subcore_axis_name="s")
_PARAMS = pltpu.CompilerParams(needs_layout_passes=False)
_PARAMS_B = pltpu.CompilerParams(needs_layout_passes=False,
                                 use_tc_tiling_on_sc=True)


@functools.partial(
    pl.kernel,
    mesh=_MESH,
    compiler_params=_PARAMS,
    out_type=jax.ShapeDtypeStruct((NW * TBL,), jnp.int32),
    scratch_types=[
        pltpu.VMEM((PPT,), jnp.int32),
        pltpu.VMEM((TBL,), jnp.int32),
    ],
)
def _winner_keys(ids_hbm, tbl_hbm, chunk, tbl):
    w = lax.axis_index("s") * 2 + lax.axis_index("c")

    lane = lax.iota(jnp.int32, 16)
    shift_up = jnp.minimum(lane + 1, 15)
    is_top = lane == 15

    def init(i, carry):
        tbl[pl.ds(i * 16, 16)] = jnp.full((16,), -1, jnp.int32)
        return carry

    lax.fori_loop(0, TBL // 16, init, 0)

    pix_base = w * PPT
    for k in range(KCH):
        pltpu.sync_copy(ids_hbm.at[pl.ds(k * HW + pix_base, PPT)], chunk)

        def body(j0, carry, k=k):
            for u in range(UNROLL):
                ids = chunk[pl.ds(j0 * 16 * UNROLL + u * 16, 16)]
                comp = ids * 16 + lane
                scomp, slane = plsc.sort_key_val(comp, lane)
                sid = lax.shift_right_logical(scomp, 4)
                nxt = sid.at[shift_up].get(mode="promise_in_bounds")
                last_of_group = (sid != nxt) | is_top
                key = (k * HW + pix_base + j0 * 16 * UNROLL + u * 16) + slane
                plsc.store_scatter(tbl, [sid], key, mask=last_of_group)
            return carry

        lax.fori_loop(0, VPT // UNROLL, body, 0)

    pltpu.sync_copy(tbl, tbl_hbm.at[pl.ds(w * TBL, TBL)])


@functools.partial(
    pl.kernel,
    mesh=_MESH,
    compiler_params=_PARAMS_B,
    out_type=jax.ShapeDtypeStruct((TBL, C), jnp.float32),
    scratch_types=[
        pltpu.VMEM((NW, VPW), jnp.int32),
        pltpu.VMEM((VPW,), jnp.int32),
        pltpu.VMEM((VPW,), jnp.int32),
        pltpu.VMEM((VPW, C), jnp.float32),
        pltpu.SemaphoreType.DMA,
    ],
)
def _gather_rows(tbl_hbm, img_hbm, out_hbm, tb, win, idxv, rows, sem):
    w = lax.axis_index("s") * 2 + lax.axis_index("c")
    vbase = w * VPW
    copies = [
        pltpu.async_copy(tbl_hbm.at[pl.ds(t * TBL + vbase, VPW)], tb.at[t], sem)
        for t in range(NW)
    ]
    for cp in copies:
        cp.wait()

    lane = lax.iota(jnp.int32, 16)
    miss_lanes = jnp.zeros((16,), jnp.int32)
    for v in range(VPW // 16):
        m = tb[0, pl.ds(v * 16, 16)]
        for t in range(1, NW):
            m = jnp.maximum(m, tb[t, pl.ds(v * 16, 16)])
        win[pl.ds(v * 16, 16)] = m
        hit = m >= 0
        vid = vbase + v * 16 + lane
        idxv[pl.ds(v * 16, 16)] = jnp.where(hit, lax.rem(m, HW), vid)
        real_miss = jnp.logical_and(jnp.logical_not(hit), vid < NV)
        miss_lanes = miss_lanes + jnp.where(real_miss, 1, 0)

    pltpu.async_copy(img_hbm.at[idxv], rows, sem).wait()

    n_miss = jnp.max(miss_lanes)

    @pl.when(n_miss > 0)
    def _patch_misses():
        def fix(r, carry):
            rs = jnp.full((16,), 0, jnp.int32) + r
            wk = plsc.load_gather(win, [rs])
            is_miss = wk < 0
            for cb in range(C // 16):
                ci = lane + cb * 16
                seg = plsc.load_gather(rows, [rs, ci])
                plsc.store_scatter(rows, [rs, ci],
                                   jnp.where(is_miss, 0.5, seg))
            return carry

        lax.fori_loop(0, VPW, fix, 0)

    pltpu.sync_copy(rows, out_hbm.at[pl.ds(vbase, VPW)])


def kernel(vert_ids, image_array):
    ids_flat = jnp.transpose(vert_ids, (2, 0, 1)).reshape(KCH * HW)
    img = image_array.reshape(HW, C)
    tbls = _winner_keys(ids_flat)
    padded = _gather_rows(tbls, img)
    return padded[:NV][None]


# speedup vs baseline: 29.3392x; 1.6878x over previous
import functools

import jax
import jax.numpy as jnp
from jax import lax
from jax.experimental import pallas as pl
from jax.experimental.pallas import tpu as pltpu
from jax.experimental.pallas import tpu_sc as plsc

H, W, C = 720, 1280, 512
NV = 2562
HW = H * W
KCH = 3
NW = 32
PPT = HW // NW
VPT = PPT // 16
TBL = 4096
VPW = TBL // NW
UNROLL = 4

_MESH = plsc.VectorSubcoreMesh(core_axis_name="c", subcore_axis_name="s")
_PARAMS = pltpu.CompilerParams(needs_layout_passes=False)
_PARAMS_B = pltpu.CompilerParams(needs_layout_passes=False,
                                 use_tc_tiling_on_sc=True)


@functools.partial(
    pl.kernel,
    mesh=_MESH,
    compiler_params=_PARAMS,
    out_type=jax.ShapeDtypeStruct((NW * TBL,), jnp.int32),
    scratch_types=[
        pltpu.VMEM((PPT,), jnp.int32),
        pltpu.VMEM((TBL,), jnp.int32),
    ],
)
def _winner_keys(ids_hbm, tbl_hbm, chunk, tbl):
    w = lax.axis_index("s") * 2 + lax.axis_index("c")

    lane = lax.iota(jnp.int32, 16)
    shift_up = jnp.minimum(lane + 1, 15)
    is_top = lane == 15

    def init(i, carry):
        tbl[pl.ds(i * 16, 16)] = jnp.full((16,), -1, jnp.int32)
        return carry

    lax.fori_loop(0, TBL // 16, init, 0)

    pix_base = w * PPT
    for k in range(KCH):
        pltpu.sync_copy(ids_hbm.at[pl.ds(k * HW + pix_base, PPT)], chunk)

        def body(j0, carry, k=k):
            for u in range(UNROLL):
                ids = chunk[pl.ds(j0 * 16 * UNROLL + u * 16, 16)]
                key = (k * HW + pix_base + j0 * 16 * UNROLL + u * 16) + lane
                plsc.store_scatter(tbl, [ids], key)
            return carry

        lax.fori_loop(0, VPT // UNROLL, body, 0)

    pltpu.sync_copy(tbl, tbl_hbm.at[pl.ds(w * TBL, TBL)])


@functools.partial(
    pl.kernel,
    mesh=_MESH,
    compiler_params=_PARAMS_B,
    out_type=jax.ShapeDtypeStruct((TBL, C), jnp.float32),
    scratch_types=[
        pltpu.VMEM((NW, VPW), jnp.int32),
        pltpu.VMEM((VPW,), jnp.int32),
        pltpu.VMEM((VPW,), jnp.int32),
        pltpu.VMEM((VPW, C), jnp.float32),
        pltpu.SemaphoreType.DMA,
    ],
)
def _gather_rows(tbl_hbm, img_hbm, out_hbm, tb, win, idxv, rows, sem):
    w = lax.axis_index("s") * 2 + lax.axis_index("c")
    vbase = w * VPW
    copies = [
        pltpu.async_copy(tbl_hbm.at[pl.ds(t * TBL + vbase, VPW)], tb.at[t], sem)
        for t in range(NW)
    ]
    for cp in copies:
        cp.wait()

    lane = lax.iota(jnp.int32, 16)
    miss_lanes = jnp.zeros((16,), jnp.int32)
    for v in range(VPW // 16):
        m = tb[0, pl.ds(v * 16, 16)]
        for t in range(1, NW):
            m = jnp.maximum(m, tb[t, pl.ds(v * 16, 16)])
        win[pl.ds(v * 16, 16)] = m
        hit = m >= 0
        vid = vbase + v * 16 + lane
        idxv[pl.ds(v * 16, 16)] = jnp.where(hit, lax.rem(m, HW), vid)
        real_miss = jnp.logical_and(jnp.logical_not(hit), vid < NV)
        miss_lanes = miss_lanes + jnp.where(real_miss, 1, 0)

    pltpu.async_copy(img_hbm.at[idxv], rows, sem).wait()

    n_miss = jnp.max(miss_lanes)

    @pl.when(n_miss > 0)
    def _patch_misses():
        def fix(r, carry):
            rs = jnp.full((16,), 0, jnp.int32) + r
            wk = plsc.load_gather(win, [rs])
            is_miss = wk < 0
            for cb in range(C // 16):
                ci = lane + cb * 16
                seg = plsc.load_gather(rows, [rs, ci])
                plsc.store_scatter(rows, [rs, ci],
                                   jnp.where(is_miss, 0.5, seg))
            return carry

        lax.fori_loop(0, VPW, fix, 0)

    pltpu.sync_copy(rows, out_hbm.at[pl.ds(vbase, VPW)])


def kernel(vert_ids, image_array):
    ids_flat = jnp.transpose(vert_ids, (2, 0, 1)).reshape(KCH * HW)
    img = image_array.reshape(HW, C)
    tbls = _winner_keys(ids_flat)
    padded = _gather_rows(tbls, img)
    return padded[:NV][None]


# speedup vs baseline: 35.3653x; 1.2054x over previous
import functools

import jax
import jax.numpy as jnp
from jax import lax
from jax.experimental import pallas as pl
from jax.experimental.pallas import tpu as pltpu
from jax.experimental.pallas import tpu_sc as plsc

H, W, C = 720, 1280, 512
NV = 2562
HW = H * W
KCH = 3
NW = 32
PPT = HW // NW
VPT = PPT // 16
TBL = 4096
VPW = TBL // NW
UNROLL = 4
FULL_TILES = NV // VPW
REM = NV - FULL_TILES * VPW

_MESH = plsc.VectorSubcoreMesh(core_axis_name="c", subcore_axis_name="s")
_PARAMS = pltpu.CompilerParams(needs_layout_passes=False)
_PARAMS_B = pltpu.CompilerParams(needs_layout_passes=False,
                                 use_tc_tiling_on_sc=True)


@functools.partial(
    pl.kernel,
    mesh=_MESH,
    compiler_params=_PARAMS,
    out_type=jax.ShapeDtypeStruct((NW * TBL,), jnp.int32),
    scratch_types=[
        pltpu.VMEM((PPT,), jnp.int32),
        pltpu.VMEM((TBL,), jnp.int32),
    ],
)
def _winner_keys(ids_hbm, tbl_hbm, chunk, tbl):
    w = lax.axis_index("s") * 2 + lax.axis_index("c")

    lane = lax.iota(jnp.int32, 16)
    shift_up = jnp.minimum(lane + 1, 15)
    is_top = lane == 15

    def init(i, carry):
        tbl[pl.ds(i * 16, 16)] = jnp.full((16,), -1, jnp.int32)
        return carry

    lax.fori_loop(0, TBL // 16, init, 0)

    pix_base = w * PPT
    for k in range(KCH):
        pltpu.sync_copy(ids_hbm.at[pl.ds(k * HW + pix_base, PPT)], chunk)

        def body(j0, carry, k=k):
            for u in range(UNROLL):
                ids = chunk[pl.ds(j0 * 16 * UNROLL + u * 16, 16)]
                key = (k * HW + pix_base + j0 * 16 * UNROLL + u * 16) + lane
                plsc.store_scatter(tbl, [ids], key)
            return carry

        lax.fori_loop(0, VPT // UNROLL, body, 0)

    pltpu.sync_copy(tbl, tbl_hbm.at[pl.ds(w * TBL, TBL)])


@functools.partial(
    pl.kernel,
    mesh=_MESH,
    compiler_params=_PARAMS_B,
    out_type=jax.ShapeDtypeStruct((1, NV, C), jnp.float32),
    scratch_types=[
        pltpu.VMEM((NW, VPW), jnp.int32),
        pltpu.VMEM((VPW,), jnp.int32),
        pltpu.VMEM((VPW,), jnp.int32),
        pltpu.VMEM((VPW, C), jnp.float32),
        pltpu.SemaphoreType.DMA,
    ],
)
def _gather_rows(tbl_hbm, img_hbm, out_hbm, tb, win, idxv, rows, sem):
    w = lax.axis_index("s") * 2 + lax.axis_index("c")
    vbase = w * VPW
    copies = [
        pltpu.async_copy(tbl_hbm.at[pl.ds(t * TBL + vbase, VPW)], tb.at[t], sem)
        for t in range(NW)
    ]
    for cp in copies:
        cp.wait()

    lane = lax.iota(jnp.int32, 16)
    miss_lanes = jnp.zeros((16,), jnp.int32)
    for v in range(VPW // 16):
        m = tb[0, pl.ds(v * 16, 16)]
        for t in range(1, NW):
            m = jnp.maximum(m, tb[t, pl.ds(v * 16, 16)])
        win[pl.ds(v * 16, 16)] = m
        hit = m >= 0
        vid = vbase + v * 16 + lane
        idxv[pl.ds(v * 16, 16)] = jnp.where(hit, lax.rem(m, HW), vid)
        real_miss = jnp.logical_and(jnp.logical_not(hit), vid < NV)
        miss_lanes = miss_lanes + jnp.where(real_miss, 1, 0)

    pltpu.async_copy(img_hbm.at[idxv], rows, sem).wait()

    n_miss = jnp.max(miss_lanes)

    @pl.when(n_miss > 0)
    def _patch_misses():
        def fix(r, carry):
            rs = jnp.full((16,), 0, jnp.int32) + r
            wk = plsc.load_gather(win, [rs])
            is_miss = wk < 0
            for cb in range(C // 16):
                ci = lane + cb * 16
                seg = plsc.load_gather(rows, [rs, ci])
                plsc.store_scatter(rows, [rs, ci],
                                   jnp.where(is_miss, 0.5, seg))
            return carry

        lax.fori_loop(0, VPW, fix, 0)

    @pl.when(w < FULL_TILES)
    def _store_full():
        pltpu.sync_copy(rows, out_hbm.at[0, pl.ds(vbase, VPW)])

    if REM:
        @pl.when(w == FULL_TILES)
        def _store_rem():
            pltpu.sync_copy(rows.at[pl.ds(0, REM)],
                            out_hbm.at[0, pl.ds(FULL_TILES * VPW, REM)])


def kernel(vert_ids, image_array):
    ids_flat = jnp.transpose(vert_ids, (2, 0, 1)).reshape(KCH * HW)
    img = image_array.reshape(HW, C)
    tbls = _winner_keys(ids_flat)
    return _gather_rows(tbls, img)


# speedup vs baseline: 37.1701x; 1.0510x over previous
import functools

import jax
import jax.numpy as jnp
from jax import lax
from jax.experimental import pallas as pl
from jax.experimental.pallas import tpu as pltpu
from jax.experimental.pallas import tpu_sc as plsc

H, W, C = 720, 1280, 512
NV = 2562
HW = H * W
KCH = 3
NW = 32
PPT = HW // NW
VPT = PPT // 16
TBL = 4096
VPW = TBL // NW
UNROLL = 4
FULL_TILES = NV // VPW
REM = NV - FULL_TILES * VPW

_MESH = plsc.VectorSubcoreMesh(core_axis_name="c", subcore_axis_name="s")
_PARAMS = pltpu.CompilerParams(needs_layout_passes=False)
_PARAMS_B = pltpu.CompilerParams(needs_layout_passes=False,
                                 use_tc_tiling_on_sc=True)


SLAB = 32
ROWS = 24


@functools.partial(
    pl.kernel,
    mesh=_MESH,
    compiler_params=_PARAMS_B,
    out_type=jax.ShapeDtypeStruct((NW * TBL,), jnp.int32),
    scratch_types=[
        pltpu.VMEM((SLAB, W), jnp.int32),
        pltpu.VMEM((TBL,), jnp.int32),
    ],
)
def _winner_keys(ids_hbm, tbl_hbm, slab, tbl):
    w = lax.axis_index("s") * 2 + lax.axis_index("c")

    lane = lax.iota(jnp.int32, 16)

    def init(i, carry):
        tbl[pl.ds(i * 16, 16)] = jnp.full((16,), -1, jnp.int32)
        return carry

    lax.fori_loop(0, TBL // 16, init, 0)

    h0 = (w * PPT) // W
    rs = jnp.minimum((h0 // 8) * 8, H - SLAB)
    n_rows = jnp.minimum(ROWS, H - h0)
    for k in range(KCH):
        pltpu.sync_copy(ids_hbm.at[pl.ds(k * H + rs, SLAB), :], slab)

        def row_body(r, carry, k=k):
            h = h0 + r
            local = h - rs
            key0 = (k * HW + h * W) + lane

            def col_body(c0, carry2):
                for u in range(UNROLL):
                    col = c0 * 16 * UNROLL + u * 16
                    ids = slab[local, pl.ds(col, 16)]
                    plsc.store_scatter(tbl, [ids], key0 + col)
                return carry2

            lax.fori_loop(0, (W // 16) // UNROLL, col_body, 0)
            return carry

        lax.fori_loop(0, n_rows, row_body, 0)

    pltpu.sync_copy(tbl, tbl_hbm.at[pl.ds(w * TBL, TBL)])


@functools.partial(
    pl.kernel,
    mesh=_MESH,
    compiler_params=_PARAMS_B,
    out_type=jax.ShapeDtypeStruct((1, NV, C), jnp.float32),
    scratch_types=[
        pltpu.VMEM((NW, VPW), jnp.int32),
        pltpu.VMEM((VPW,), jnp.int32),
        pltpu.VMEM((VPW,), jnp.int32),
        pltpu.VMEM((VPW, C), jnp.float32),
        pltpu.SemaphoreType.DMA,
    ],
)
def _gather_rows(tbl_hbm, img_hbm, out_hbm, tb, win, idxv, rows, sem):
    w = lax.axis_index("s") * 2 + lax.axis_index("c")
    vbase = w * VPW
    copies = [
        pltpu.async_copy(tbl_hbm.at[pl.ds(t * TBL + vbase, VPW)], tb.at[t], sem)
        for t in range(NW)
    ]
    for cp in copies:
        cp.wait()

    lane = lax.iota(jnp.int32, 16)
    miss_lanes = jnp.zeros((16,), jnp.int32)
    for v in range(VPW // 16):
        m = tb[0, pl.ds(v * 16, 16)]
        for t in range(1, NW):
            m = jnp.maximum(m, tb[t, pl.ds(v * 16, 16)])
        win[pl.ds(v * 16, 16)] = m
        hit = m >= 0
        vid = vbase + v * 16 + lane
        idxv[pl.ds(v * 16, 16)] = jnp.where(hit, lax.rem(m, HW), vid)
        real_miss = jnp.logical_and(jnp.logical_not(hit), vid < NV)
        miss_lanes = miss_lanes + jnp.where(real_miss, 1, 0)

    pltpu.async_copy(img_hbm.at[idxv], rows, sem).wait()

    n_miss = jnp.max(miss_lanes)

    @pl.when(n_miss > 0)
    def _patch_misses():
        def fix(r, carry):
            rs = jnp.full((16,), 0, jnp.int32) + r
            wk = plsc.load_gather(win, [rs])
            is_miss = wk < 0
            for cb in range(C // 16):
                ci = lane + cb * 16
                seg = plsc.load_gather(rows, [rs, ci])
                plsc.store_scatter(rows, [rs, ci],
                                   jnp.where(is_miss, 0.5, seg))
            return carry

        lax.fori_loop(0, VPW, fix, 0)

    @pl.when(w < FULL_TILES)
    def _store_full():
        pltpu.sync_copy(rows, out_hbm.at[0, pl.ds(vbase, VPW)])

    if REM:
        @pl.when(w == FULL_TILES)
        def _store_rem():
            pltpu.sync_copy(rows.at[pl.ds(0, REM)],
                            out_hbm.at[0, pl.ds(FULL_TILES * VPW, REM)])


def kernel(vert_ids, image_array):
    ids2d = jnp.transpose(vert_ids, (2, 0, 1)).reshape(KCH * H, W)
    img = image_array.reshape(HW, C)
    tbls = _winner_keys(ids2d)
    return _gather_rows(tbls, img)


# speedup vs baseline: 39.8525x; 1.0722x over previous
import functools

import jax
import jax.numpy as jnp
from jax import lax
from jax.experimental import pallas as pl
from jax.experimental.pallas import tpu as pltpu
from jax.experimental.pallas import tpu_sc as plsc

H, W, C = 720, 1280, 512
NV = 2562
HW = H * W
KCH = 3
NW = 32
PPT = HW // NW
VPT = PPT // 16
TBL = 4096
VPW = TBL // NW
UNROLL = 8
FULL_TILES = NV // VPW
REM = NV - FULL_TILES * VPW

_MESH = plsc.VectorSubcoreMesh(core_axis_name="c", subcore_axis_name="s")
_PARAMS = pltpu.CompilerParams(needs_layout_passes=False)
_PARAMS_B = pltpu.CompilerParams(needs_layout_passes=False,
                                 use_tc_tiling_on_sc=True)


SLAB = 32
ROWS = 24


@functools.partial(
    pl.kernel,
    mesh=_MESH,
    compiler_params=_PARAMS_B,
    out_type=jax.ShapeDtypeStruct((NW * TBL,), jnp.int32),
    scratch_types=[
        pltpu.VMEM((2, SLAB, W), jnp.int32),
        pltpu.VMEM((TBL,), jnp.int32),
        pltpu.SemaphoreType.DMA,
    ],
)
def _winner_keys(ids_hbm, tbl_hbm, slabs, tbl, sem):
    w = lax.axis_index("s") * 2 + lax.axis_index("c")

    lane = lax.iota(jnp.int32, 16)

    def init(i, carry):
        tbl[pl.ds(i * 16, 16)] = jnp.full((16,), -1, jnp.int32)
        return carry

    lax.fori_loop(0, TBL // 16, init, 0)

    h0 = (w * PPT) // W
    rs = jnp.minimum((h0 // 8) * 8, H - SLAB)
    n_rows = jnp.minimum(ROWS, H - h0)
    pending = pltpu.async_copy(ids_hbm.at[pl.ds(rs, SLAB), :], slabs.at[0], sem)
    for k in range(KCH):
        pending.wait()
        if k + 1 < KCH:
            pending = pltpu.async_copy(
                ids_hbm.at[pl.ds((k + 1) * H + rs, SLAB), :],
                slabs.at[(k + 1) % 2], sem)
        slab = slabs.at[k % 2]

        def row_body(r, carry, k=k, slab=slab):
            h = h0 + r
            local = h - rs
            key0 = (k * HW + h * W) + lane

            def col_body(c0, carry2):
                for u in range(UNROLL):
                    col = c0 * 16 * UNROLL + u * 16
                    ids = slab[local, pl.ds(col, 16)]
                    plsc.store_scatter(tbl, [ids], key0 + col)
                return carry2

            lax.fori_loop(0, (W // 16) // UNROLL, col_body, 0)
            return carry

        lax.fori_loop(0, n_rows, row_body, 0)

    pltpu.sync_copy(tbl, tbl_hbm.at[pl.ds(w * TBL, TBL)])


@functools.partial(
    pl.kernel,
    mesh=_MESH,
    compiler_params=_PARAMS_B,
    out_type=jax.ShapeDtypeStruct((1, NV, C), jnp.float32),
    scratch_types=[
        pltpu.VMEM((NW, VPW), jnp.int32),
        pltpu.VMEM((VPW,), jnp.int32),
        pltpu.VMEM((VPW,), jnp.int32),
        pltpu.VMEM((VPW, C), jnp.float32),
        pltpu.SemaphoreType.DMA,
    ],
)
def _gather_rows(tbl_hbm, img_hbm, out_hbm, tb, win, idxv, rows, sem):
    w = lax.axis_index("s") * 2 + lax.axis_index("c")
    vbase = w * VPW
    copies = [
        pltpu.async_copy(tbl_hbm.at[pl.ds(t * TBL + vbase, VPW)], tb.at[t], sem)
        for t in range(NW)
    ]
    for cp in copies:
        cp.wait()

    lane = lax.iota(jnp.int32, 16)
    miss_lanes = jnp.zeros((16,), jnp.int32)
    for v in range(VPW // 16):
        m = tb[0, pl.ds(v * 16, 16)]
        for t in range(1, NW):
            m = jnp.maximum(m, tb[t, pl.ds(v * 16, 16)])
        win[pl.ds(v * 16, 16)] = m
        hit = m >= 0
        vid = vbase + v * 16 + lane
        idxv[pl.ds(v * 16, 16)] = jnp.where(hit, lax.rem(m, HW), vid)
        real_miss = jnp.logical_and(jnp.logical_not(hit), vid < NV)
        miss_lanes = miss_lanes + jnp.where(real_miss, 1, 0)

    pltpu.async_copy(img_hbm.at[idxv], rows, sem).wait()

    n_miss = jnp.max(miss_lanes)

    @pl.when(n_miss > 0)
    def _patch_misses():
        def fix(r, carry):
            rs = jnp.full((16,), 0, jnp.int32) + r
            wk = plsc.load_gather(win, [rs])
            is_miss = wk < 0
            for cb in range(C // 16):
                ci = lane + cb * 16
                seg = plsc.load_gather(rows, [rs, ci])
                plsc.store_scatter(rows, [rs, ci],
                                   jnp.where(is_miss, 0.5, seg))
            return carry

        lax.fori_loop(0, VPW, fix, 0)

    @pl.when(w < FULL_TILES)
    def _store_full():
        pltpu.sync_copy(rows, out_hbm.at[0, pl.ds(vbase, VPW)])

    if REM:
        @pl.when(w == FULL_TILES)
        def _store_rem():
            pltpu.sync_copy(rows.at[pl.ds(0, REM)],
                            out_hbm.at[0, pl.ds(FULL_TILES * VPW, REM)])


def kernel(vert_ids, image_array):
    ids2d = jnp.transpose(vert_ids, (2, 0, 1)).reshape(KCH * H, W)
    img = image_array.reshape(HW, C)
    tbls = _winner_keys(ids2d)
    return _gather_rows(tbls, img)


# speedup vs baseline: 39.9171x; 1.0016x over previous
import functools

import jax
import jax.numpy as jnp
from jax import lax
from jax.experimental import pallas as pl
from jax.experimental.pallas import tpu as pltpu
from jax.experimental.pallas import tpu_sc as plsc

H, W, C = 720, 1280, 512
NV = 2562
HW = H * W
KCH = 3
NW = 32
PPT = HW // NW
VPT = PPT // 16
TBL = 4096
VPW = TBL // NW
UNROLL = 8
FULL_TILES = NV // VPW
REM = NV - FULL_TILES * VPW

_MESH = plsc.VectorSubcoreMesh(core_axis_name="c", subcore_axis_name="s")
_PARAMS = pltpu.CompilerParams(needs_layout_passes=False)
_PARAMS_B = pltpu.CompilerParams(needs_layout_passes=False,
                                 use_tc_tiling_on_sc=True)


SLAB = 32
ROWS = 24


@functools.partial(
    pl.kernel,
    mesh=_MESH,
    compiler_params=_PARAMS_B,
    out_type=jax.ShapeDtypeStruct((NW * TBL,), jnp.int32),
    scratch_types=[
        pltpu.VMEM((2, SLAB, W), jnp.int32),
        pltpu.VMEM((TBL,), jnp.int32),
        pltpu.SemaphoreType.DMA,
    ],
)
def _winner_keys(ids_hbm, tbl_hbm, slabs, tbl, sem):
    w = lax.axis_index("s") * 2 + lax.axis_index("c")

    lane = lax.iota(jnp.int32, 16)

    def init(i, carry):
        tbl[pl.ds(i * 16, 16)] = jnp.full((16,), -1, jnp.int32)
        return carry

    lax.fori_loop(0, TBL // 16, init, 0)

    h0 = (w * PPT) // W
    rs = jnp.minimum((h0 // 8) * 8, H - SLAB)
    n_rows = jnp.minimum(ROWS, H - h0)
    pending = pltpu.async_copy(ids_hbm.at[pl.ds(rs, SLAB), :], slabs.at[0], sem)
    for k in range(KCH):
        pending.wait()
        if k + 1 < KCH:
            pending = pltpu.async_copy(
                ids_hbm.at[pl.ds((k + 1) * H + rs, SLAB), :],
                slabs.at[(k + 1) % 2], sem)
        slab = slabs.at[k % 2]

        def row_body(r, carry, k=k, slab=slab):
            h = h0 + r
            local = h - rs
            key0 = (k * HW + h * W) + lane

            def col_body(c0, carry2):
                for u in range(UNROLL):
                    col = c0 * 16 * UNROLL + u * 16
                    ids = slab[local, pl.ds(col, 16)]
                    plsc.store_scatter(tbl, [ids], key0 + col)
                return carry2

            lax.fori_loop(0, (W // 16) // UNROLL, col_body, 0)
            return carry

        lax.fori_loop(0, n_rows, row_body, 0)

    pltpu.sync_copy(tbl, tbl_hbm.at[pl.ds(w * TBL, TBL)])


@functools.partial(
    pl.kernel,
    mesh=_MESH,
    compiler_params=_PARAMS_B,
    out_type=jax.ShapeDtypeStruct((1, NV, C), jnp.float32),
    scratch_types=[
        pltpu.VMEM((NW, VPW), jnp.int32),
        pltpu.VMEM((VPW,), jnp.int32),
        pltpu.VMEM((VPW,), jnp.int32),
        pltpu.VMEM((VPW, C), jnp.float32),
        pltpu.SemaphoreType.DMA,
    ],
)
def _gather_rows(tbl_hbm, img_hbm, out_hbm, tb, win, idxv, rows, sem):
    w = lax.axis_index("s") * 2 + lax.axis_index("c")
    vbase = w * VPW

    @pl.when(w <= FULL_TILES)
    def _tile_work():
        _gather_tile(tbl_hbm, img_hbm, out_hbm, tb, win, idxv, rows, sem,
                     w, vbase)


def _gather_tile(tbl_hbm, img_hbm, out_hbm, tb, win, idxv, rows, sem,
                 w, vbase):
    copies = [
        pltpu.async_copy(tbl_hbm.at[pl.ds(t * TBL + vbase, VPW)], tb.at[t], sem)
        for t in range(NW)
    ]
    for cp in copies:
        cp.wait()

    lane = lax.iota(jnp.int32, 16)
    miss_lanes = jnp.zeros((16,), jnp.int32)
    for v in range(VPW // 16):
        m = tb[0, pl.ds(v * 16, 16)]
        for t in range(1, NW):
            m = jnp.maximum(m, tb[t, pl.ds(v * 16, 16)])
        win[pl.ds(v * 16, 16)] = m
        hit = m >= 0
        vid = vbase + v * 16 + lane
        idxv[pl.ds(v * 16, 16)] = jnp.where(hit, lax.rem(m, HW), vid)
        real_miss = jnp.logical_and(jnp.logical_not(hit), vid < NV)
        miss_lanes = miss_lanes + jnp.where(real_miss, 1, 0)

    pltpu.async_copy(img_hbm.at[idxv], rows, sem).wait()

    n_miss = jnp.max(miss_lanes)

    @pl.when(n_miss > 0)
    def _patch_misses():
        def fix(r, carry):
            rs = jnp.full((16,), 0, jnp.int32) + r
            wk = plsc.load_gather(win, [rs])
            is_miss = wk < 0
            for cb in range(C // 16):
                ci = lane + cb * 16
                seg = plsc.load_gather(rows, [rs, ci])
                plsc.store_scatter(rows, [rs, ci],
                                   jnp.where(is_miss, 0.5, seg))
            return carry

        lax.fori_loop(0, VPW, fix, 0)

    @pl.when(w < FULL_TILES)
    def _store_full():
        pltpu.sync_copy(rows, out_hbm.at[0, pl.ds(vbase, VPW)])

    if REM:
        @pl.when(w == FULL_TILES)
        def _store_rem():
            pltpu.sync_copy(rows.at[pl.ds(0, REM)],
                            out_hbm.at[0, pl.ds(FULL_TILES * VPW, REM)])


def kernel(vert_ids, image_array):
    ids2d = jnp.transpose(vert_ids, (2, 0, 1)).reshape(KCH * H, W)
    img = image_array.reshape(HW, C)
    tbls = _winner_keys(ids2d)
    return _gather_rows(tbls, img)


# speedup vs baseline: 55.3438x; 1.3865x over previous
import functools

import jax
import jax.numpy as jnp
from jax import lax
from jax.experimental import pallas as pl
from jax.experimental.pallas import tpu as pltpu
from jax.experimental.pallas import tpu_sc as plsc

H, W, C = 720, 1280, 512
NV = 2562
HW = H * W
KCH = 3
NW = 32
PPT = HW // NW
VPT = PPT // 16
TBL = 4096
VPW = TBL // NW
UNROLL = 8
FULL_TILES = NV // VPW
REM = NV - FULL_TILES * VPW

_MESH = plsc.VectorSubcoreMesh(core_axis_name="c", subcore_axis_name="s")
_PARAMS = pltpu.CompilerParams(needs_layout_passes=False)
_PARAMS_B = pltpu.CompilerParams(needs_layout_passes=False,
                                 use_tc_tiling_on_sc=True)


SLAB = 32
ROWS = 24


@functools.partial(
    pl.kernel,
    mesh=_MESH,
    compiler_params=_PARAMS_B,
    out_type=jax.ShapeDtypeStruct((NW * TBL,), jnp.int32),
    scratch_types=[
        pltpu.VMEM((2, SLAB, W), jnp.int32),
        pltpu.VMEM((TBL,), jnp.int32),
        pltpu.SemaphoreType.DMA,
    ],
)
def _winner_keys(ids_hbm, tbl_hbm, slabs, tbl, sem):
    w = lax.axis_index("s") * 2 + lax.axis_index("c")

    lane = lax.iota(jnp.int32, 16)

    def init(i, carry):
        tbl[pl.ds(i * 16, 16)] = jnp.full((16,), -1, jnp.int32)
        return carry

    lax.fori_loop(0, TBL // 16, init, 0)

    h0 = (w * PPT) // W
    rs = jnp.minimum((h0 // 8) * 8, H - SLAB)
    n_rows = jnp.minimum(ROWS, H - h0)
    pending = pltpu.async_copy(ids_hbm.at[pl.ds(rs, SLAB), :], slabs.at[0], sem)
    for k in range(KCH):
        pending.wait()
        if k + 1 < KCH:
            pending = pltpu.async_copy(
                ids_hbm.at[pl.ds((k + 1) * H + rs, SLAB), :],
                slabs.at[(k + 1) % 2], sem)
        slab = slabs.at[k % 2]

        def row_body(r, carry, k=k, slab=slab):
            h = h0 + r
            local = h - rs
            key0 = (k * HW + h * W) + lane

            def col_body(c0, carry2):
                cols = [c0 * 16 * UNROLL + u * 16 for u in range(UNROLL)]
                ids = [slab[local, pl.ds(col, 16)] for col in cols]
                for u, col in enumerate(cols):
                    plsc.store_scatter(tbl, [ids[u]], key0 + col)
                return carry2

            lax.fori_loop(0, (W // 16) // UNROLL, col_body, 0)
            return carry

        lax.fori_loop(0, n_rows, row_body, 0)

    pltpu.sync_copy(tbl, tbl_hbm.at[pl.ds(w * TBL, TBL)])


@functools.partial(
    pl.kernel,
    mesh=_MESH,
    compiler_params=_PARAMS_B,
    out_type=jax.ShapeDtypeStruct((1, NV, C), jnp.float32),
    scratch_types=[
        pltpu.VMEM((NW, VPW), jnp.int32),
        pltpu.VMEM((VPW,), jnp.int32),
        pltpu.VMEM((VPW,), jnp.int32),
        pltpu.VMEM((VPW, C), jnp.float32),
        pltpu.SemaphoreType.DMA,
    ],
)
def _gather_rows(tbl_hbm, img_hbm, out_hbm, tb, win, idxv, rows, sem):
    w = lax.axis_index("s") * 2 + lax.axis_index("c")
    vbase = w * VPW

    @pl.when(w <= FULL_TILES)
    def _tile_work():
        _gather_tile(tbl_hbm, img_hbm, out_hbm, tb, win, idxv, rows, sem,
                     w, vbase)


def _gather_tile(tbl_hbm, img_hbm, out_hbm, tb, win, idxv, rows, sem,
                 w, vbase):
    copies = [
        pltpu.async_copy(tbl_hbm.at[pl.ds(t * TBL + vbase, VPW)], tb.at[t], sem)
        for t in range(NW)
    ]
    for cp in copies:
        cp.wait()

    lane = lax.iota(jnp.int32, 16)
    miss_lanes = jnp.zeros((16,), jnp.int32)
    for v in range(VPW // 16):
        m = tb[0, pl.ds(v * 16, 16)]
        for t in range(1, NW):
            m = jnp.maximum(m, tb[t, pl.ds(v * 16, 16)])
        win[pl.ds(v * 16, 16)] = m
        hit = m >= 0
        vid = vbase + v * 16 + lane
        idxv[pl.ds(v * 16, 16)] = jnp.where(hit, lax.rem(m, HW), vid)
        real_miss = jnp.logical_and(jnp.logical_not(hit), vid < NV)
        miss_lanes = miss_lanes + jnp.where(real_miss, 1, 0)

    pltpu.async_copy(img_hbm.at[idxv], rows, sem).wait()

    n_miss = jnp.max(miss_lanes)

    @pl.when(n_miss > 0)
    def _patch_misses():
        def fix(r, carry):
            rs = jnp.full((16,), 0, jnp.int32) + r
            wk = plsc.load_gather(win, [rs])
            is_miss = wk < 0
            for cb in range(C // 16):
                ci = lane + cb * 16
                seg = plsc.load_gather(rows, [rs, ci])
                plsc.store_scatter(rows, [rs, ci],
                                   jnp.where(is_miss, 0.5, seg))
            return carry

        lax.fori_loop(0, VPW, fix, 0)

    @pl.when(w < FULL_TILES)
    def _store_full():
        pltpu.sync_copy(rows, out_hbm.at[0, pl.ds(vbase, VPW)])

    if REM:
        @pl.when(w == FULL_TILES)
        def _store_rem():
            pltpu.sync_copy(rows.at[pl.ds(0, REM)],
                            out_hbm.at[0, pl.ds(FULL_TILES * VPW, REM)])


def kernel(vert_ids, image_array):
    ids2d = jnp.transpose(vert_ids, (2, 0, 1)).reshape(KCH * H, W)
    img = image_array.reshape(HW, C)
    tbls = _winner_keys(ids2d)
    return _gather_rows(tbls, img)
